# Initial kernel scaffold; baseline (speedup 1.0000x reference)
import jax
import jax.numpy as jnp
from jax import lax
from jax.experimental import pallas as pl
from jax.experimental.pallas import tpu as pltpu
from jax.experimental.pallas import tpu_sc as plsc

N_NODES = 50000
HIDDEN = 256
GRAPHS = 512
OUT = 128

NC = 2
NS = 16
NW = NC * NS
CHUNK = 112
NCHUNK = 14
ROWS_PER_W = CHUNK * NCHUNK
N_PAD = ROWS_PER_W * NW
LANES = 16


def _sc_body(x_hbm, idx_hbm, out_hbm, idx_v, rows_v, zbuf, acc):
    c = lax.axis_index("c")
    s = lax.axis_index("s")
    wid = s * NC + c

    def zero_row(i, carry):
        for j in range(HIDDEN // LANES):
            zbuf[i, pl.ds(j * LANES, LANES)] = jnp.zeros((LANES,), jnp.float32)
        return carry

    lax.fori_loop(0, GRAPHS // NS, zero_row, 0)
    pltpu.sync_copy(zbuf, acc.at[pl.ds(s * (GRAPHS // NS), GRAPHS // NS)])
    plsc.subcore_barrier()

    pltpu.sync_copy(idx_hbm.at[wid], idx_v)

    base = wid * ROWS_PER_W
    for j in range(NCHUNK):
        pltpu.sync_copy(x_hbm.at[pl.ds(base + j * CHUNK, CHUNK)], rows_v)
        pltpu.sync_copy(rows_v, acc.at[idx_v.at[j]], add=True)

    plsc.subcore_barrier()
    pltpu.sync_copy(
        acc.at[pl.ds(s * (GRAPHS // NS), GRAPHS // NS)],
        out_hbm.at[c].at[pl.ds(s * (GRAPHS // NS), GRAPHS // NS)],
    )


def _segment_sum_sc(x_pad, idx_resh):
    mesh = plsc.VectorSubcoreMesh(
        core_axis_name="c", subcore_axis_name="s", num_cores=NC, num_subcores=NS
    )
    return pl.kernel(
        _sc_body,
        out_type=jax.ShapeDtypeStruct((NC, GRAPHS, HIDDEN), jnp.float32),
        mesh=mesh,
        scratch_types=[
            pltpu.VMEM((NCHUNK, CHUNK), jnp.int32),
            pltpu.VMEM((CHUNK, HIDDEN), jnp.float32),
            pltpu.VMEM((GRAPHS // NS, HIDDEN), jnp.float32),
            pltpu.VMEM_SHARED((GRAPHS, HIDDEN), jnp.float32),
        ],
    )(x_pad, idx_resh)


def _tc_body(p_ref, w_ref, b_ref, o_ref):
    pooled = p_ref[0] + p_ref[1]
    o_ref[...] = (
        lax.dot_general(
            pooled, w_ref[...], (((1,), (1,)), ((), ())),
            preferred_element_type=jnp.float32,
        )
        + b_ref[...]
    )


def _linear_tc(parts, W, b):
    return pl.pallas_call(
        _tc_body,
        out_shape=jax.ShapeDtypeStruct((GRAPHS, OUT), jnp.float32),
    )(parts, W, b.reshape(1, OUT))


def kernel(x, batch, W, b):
    idx = batch.astype(jnp.int32)
    pad = N_PAD - N_NODES
    x_pad = jnp.concatenate([x, jnp.zeros((pad, HIDDEN), x.dtype)], axis=0)
    idx_pad = jnp.concatenate([idx, jnp.full((pad,), GRAPHS - 1, jnp.int32)])
    idx_resh = idx_pad.reshape(NW, NCHUNK, CHUNK)
    parts = _segment_sum_sc(x_pad, idx_resh)
    return _linear_tc(parts, W, b)


# speedup vs baseline: 1.6971x; 1.6971x over previous
import jax
import jax.numpy as jnp
from jax import lax
from jax.experimental import pallas as pl
from jax.experimental.pallas import tpu as pltpu
from jax.experimental.pallas import tpu_sc as plsc

N_NODES = 50000
HIDDEN = 256
HALF = 128
GRAPHS = 512
OUT = 128

NC = 2
NS = 16
WPH = NC * 8
CHUNK = 224
NCHUNK = 14
ROWS_PER_W = CHUNK * NCHUNK
N_PAD = ROWS_PER_W * WPH
LANES = 16
GROUPS = CHUNK // LANES


def _sc_body(x_hbm, idx_hbm, out_hbm, idx_v, rows_v, acc):
    c = lax.axis_index("c")
    s = lax.axis_index("s")
    h = s // 8
    q = c * 8 + lax.rem(s, 8)

    def zero_row(i, carry):
        acc[pl.ds(i * LANES, LANES)] = jnp.zeros((LANES,), jnp.float32)
        return carry

    lax.fori_loop(0, GRAPHS * HALF // LANES, zero_row, 0)

    pltpu.sync_copy(idx_hbm.at[q], idx_v)

    base = q * ROWS_PER_W
    for j in range(NCHUNK):
        pltpu.sync_copy(x_hbm.at[pl.ds(base + j * CHUNK, CHUNK), h], rows_v)

        def group_body(g, carry):
            seg_vec = idx_v[pl.ds(j * CHUNK + g * LANES, LANES)]
            for lane in range(LANES):
                seg = seg_vec.at[jnp.full((LANES,), lane, jnp.int32)].get(
                    mode="promise_in_bounds"
                )
                row = g * LANES + lane
                addr_base = seg * HALF
                for j8 in range(HALF // LANES):
                    addr = addr_base + (lax.iota(jnp.int32, LANES) + j8 * LANES)
                    data = rows_v[row, pl.ds(j8 * LANES, LANES)]
                    plsc.addupdate_scatter(acc, [addr], data)
            return carry

        lax.fori_loop(0, GROUPS, group_body, 0)

    pltpu.sync_copy(acc, out_hbm.at[c].at[s])


def _segment_sum_sc(x3, idx_resh):
    mesh = plsc.VectorSubcoreMesh(
        core_axis_name="c", subcore_axis_name="s", num_cores=NC, num_subcores=NS
    )
    return pl.kernel(
        _sc_body,
        out_type=jax.ShapeDtypeStruct((NC, NS, GRAPHS * HALF), jnp.float32),
        mesh=mesh,
        compiler_params=pltpu.CompilerParams(
            use_tc_tiling_on_sc=False, needs_layout_passes=False
        ),
        scratch_types=[
            pltpu.VMEM((ROWS_PER_W,), jnp.int32),
            pltpu.VMEM((CHUNK, HALF), jnp.float32),
            pltpu.VMEM((GRAPHS * HALF,), jnp.float32),
        ],
    )(x3, idx_resh)


def _tc_body(p_ref, w_ref, b_ref, o_ref):
    p = p_ref[...].reshape(NC, NS, GRAPHS, HALF)
    lo = jnp.sum(p[:, 0:8], axis=(0, 1))
    hi = jnp.sum(p[:, 8:16], axis=(0, 1))
    pooled = jnp.concatenate([lo, hi], axis=1)
    o_ref[...] = (
        lax.dot_general(
            pooled, w_ref[...], (((1,), (1,)), ((), ())),
            preferred_element_type=jnp.float32,
        )
        + b_ref[...]
    )


def _linear_tc(parts, W, b):
    return pl.pallas_call(
        _tc_body,
        out_shape=jax.ShapeDtypeStruct((GRAPHS, OUT), jnp.float32),
    )(parts, W, b.reshape(1, OUT))


def kernel(x, batch, W, b):
    idx = batch.astype(jnp.int32)
    pad = N_PAD - N_NODES
    x_pad = jnp.concatenate([x, jnp.zeros((pad, HIDDEN), x.dtype)], axis=0)
    x3 = x_pad.reshape(N_PAD, 2, HALF)
    idx_pad = jnp.concatenate([idx, jnp.full((pad,), GRAPHS - 1, jnp.int32)])
    idx_resh = idx_pad.reshape(WPH, ROWS_PER_W)
    parts = _segment_sum_sc(x3, idx_resh)
    return _linear_tc(parts, W, b)


# speedup vs baseline: 2.3026x; 1.3568x over previous
import jax
import jax.numpy as jnp
from jax import lax
from jax.experimental import pallas as pl
from jax.experimental.pallas import tpu as pltpu
from jax.experimental.pallas import tpu_sc as plsc

N_NODES = 50000
HIDDEN = 256
HALF = 128
GRAPHS = 512
OUT = 128

NC = 2
NS = 16
WPH = NC * 8
RPW = N_NODES // WPH
CHUNK = 224
NFULL = 13
TAIL = RPW - NFULL * CHUNK
LANES = 16
NVEC = HALF // LANES


def _sc_body(x_hbm, idx_hbm, out_hbm, idx_v, rows_v, acc):
    c = lax.axis_index("c")
    s = lax.axis_index("s")
    h = s // 8
    q = c * 8 + lax.rem(s, 8)

    def zero_row(i, carry):
        acc[pl.ds(i * LANES, LANES)] = jnp.zeros((LANES,), jnp.float32)
        return carry

    lax.fori_loop(0, GRAPHS * HALF // LANES, zero_row, 0)

    pltpu.sync_copy(idx_hbm.at[q], idx_v.at[pl.ds(0, RPW)])

    base = q * RPW
    zero8 = tuple(jnp.zeros((LANES,), jnp.float32) for _ in range(NVEC))
    carry = (idx_v[pl.ds(0, LANES)][0],) + zero8

    for j in range(NFULL + 1):
        rows = CHUNK if j < NFULL else TAIL
        start = j * CHUNK
        if rows == CHUNK:
            pltpu.sync_copy(x_hbm.at[pl.ds(base + start, rows), h], rows_v)
        else:
            pltpu.sync_copy(
                x_hbm.at[pl.ds(base + start, rows), h],
                rows_v.at[pl.ds(0, rows)],
            )

        def row_body(r, carry):
            prev = carry[0]
            a = carry[1:]
            seg = idx_v[pl.ds(start + r, LANES)][0]
            flush = seg != prev

            @pl.when(flush)
            def _():
                for k in range(NVEC):
                    acc[pl.ds(prev * HALF + k * LANES, LANES)] = a[k]

            fv = jnp.full((LANES,), flush)
            new_a = tuple(
                jnp.where(fv, rows_v[r, pl.ds(k * LANES, LANES)],
                          a[k] + rows_v[r, pl.ds(k * LANES, LANES)])
                for k in range(NVEC)
            )
            return (seg,) + new_a

        carry = lax.fori_loop(0, rows, row_body, carry)

    prev = carry[0]
    for k in range(NVEC):
        acc[pl.ds(prev * HALF + k * LANES, LANES)] = carry[1 + k]

    pltpu.sync_copy(acc, out_hbm.at[c].at[s])


def _segment_sum_sc(x3, idx_resh):
    mesh = plsc.VectorSubcoreMesh(
        core_axis_name="c", subcore_axis_name="s", num_cores=NC, num_subcores=NS
    )
    return pl.kernel(
        _sc_body,
        out_type=jax.ShapeDtypeStruct((NC, NS, GRAPHS * HALF), jnp.float32),
        mesh=mesh,
        compiler_params=pltpu.CompilerParams(
            use_tc_tiling_on_sc=False, needs_layout_passes=False
        ),
        scratch_types=[
            pltpu.VMEM((RPW + LANES,), jnp.int32),
            pltpu.VMEM((CHUNK, HALF), jnp.float32),
            pltpu.VMEM((GRAPHS * HALF,), jnp.float32),
        ],
    )(x3, idx_resh)


def _tc_body(p_ref, w_ref, b_ref, o_ref):
    p = p_ref[...].reshape(NC, NS, GRAPHS, HALF)
    lo = jnp.sum(p[:, 0:8], axis=(0, 1))
    hi = jnp.sum(p[:, 8:16], axis=(0, 1))
    pooled = jnp.concatenate([lo, hi], axis=1)
    o_ref[...] = (
        lax.dot_general(
            pooled, w_ref[...], (((1,), (1,)), ((), ())),
            preferred_element_type=jnp.float32,
        )
        + b_ref[...]
    )


def _linear_tc(parts, W, b):
    return pl.pallas_call(
        _tc_body,
        out_shape=jax.ShapeDtypeStruct((GRAPHS, OUT), jnp.float32),
    )(parts, W, b.reshape(1, OUT))


def kernel(x, batch, W, b):
    idx_resh = batch.astype(jnp.int32).reshape(WPH, RPW)
    x3 = x.reshape(N_NODES, 2, HALF)
    parts = _segment_sum_sc(x3, idx_resh)
    return _linear_tc(parts, W, b)


# speedup vs baseline: 4.1381x; 1.7971x over previous
import jax
import jax.numpy as jnp
from jax import lax
from jax.experimental import pallas as pl
from jax.experimental.pallas import tpu as pltpu
from jax.experimental.pallas import tpu_sc as plsc

N_NODES = 50000
HIDDEN = 256
HALF = 128
GRAPHS = 512
OUT = 128

NC = 2
NS = 16
WPH = NC * 8
NBLK = N_NODES // 8
RPB = 390
RPW = RPB * 8
BCH = 28
NFULL = 13
TAILB = RPB - NFULL * BCH
XTRA_B = NBLK - WPH * RPB
XTRA = XTRA_B * 8
LANES = 16
NVEC = HALF // LANES
IDXT = RPW


def _sc_body(x4_hbm, idx_hbm, out_hbm, idx_v, rows_a, rows_b, acc, sem_a, sem_b):
    c = lax.axis_index("c")
    s = lax.axis_index("s")
    h = s // 8
    q = c * 8 + lax.rem(s, 8)

    def zero_row(i, carry):
        acc[pl.ds(i * LANES, LANES)] = jnp.zeros((LANES,), jnp.float32)
        return carry

    lax.fori_loop(0, GRAPHS * HALF // LANES, zero_row, 0)

    pltpu.sync_copy(idx_hbm.at[pl.ds(q * RPW, RPW)], idx_v.at[pl.ds(0, RPW)])
    pltpu.sync_copy(
        idx_hbm.at[pl.ds(WPH * RPW, XTRA)], idx_v.at[pl.ds(IDXT, XTRA)]
    )

    bufs = (rows_a, rows_b)
    sems = (sem_a, sem_b)
    bbase = q * RPB

    def fetch(j):
        nb = BCH if j < NFULL else TAILB
        src = x4_hbm.at[pl.ds(bbase + j * BCH, nb), h]
        dst = bufs[j % 2] if nb == BCH else bufs[j % 2].at[pl.ds(0, nb)]
        return pltpu.async_copy(src, dst, sems[j % 2])

    zero8 = tuple(jnp.zeros((LANES,), jnp.float32) for _ in range(NVEC))
    carry = (idx_v[pl.ds(0, LANES)][0],) + zero8

    pending = fetch(0)
    for j in range(NFULL + 1):
        nb = BCH if j < NFULL else TAILB
        start = j * BCH * 8
        rows_v = bufs[j % 2]
        nxt = fetch(j + 1) if j < NFULL else None
        pending.wait()
        pending = nxt

        def row_body(r, carry):
            prev = carry[0]
            a = carry[1:]
            seg = idx_v[pl.ds(start + r, LANES)][0]
            flush = seg != prev

            @pl.when(flush)
            def _():
                for k in range(NVEC):
                    acc[pl.ds(prev * HALF + k * LANES, LANES)] = a[k]

            fv = jnp.full((LANES,), flush)
            new_a = tuple(
                jnp.where(
                    fv,
                    rows_v[r // 8, lax.rem(r, 8), pl.ds(k * LANES, LANES)],
                    a[k] + rows_v[r // 8, lax.rem(r, 8), pl.ds(k * LANES, LANES)],
                )
                for k in range(NVEC)
            )
            return (seg,) + new_a

        def group_body(g, carry):
            prev = carry[0]
            seg_vec = idx_v[pl.ds(start + g * LANES, LANES)]
            n_same = plsc.all_reduce_population_count(
                seg_vec == jnp.full((LANES,), prev, jnp.int32)
            )[0]

            def fast(carry):
                a = carry[1:]
                new_a = []
                for k in range(NVEC):
                    d = [
                        rows_v[2 * g + i // 8, i % 8, pl.ds(k * LANES, LANES)]
                        for i in range(LANES)
                    ]
                    while len(d) > 1:
                        d = [d[i] + d[i + 1] for i in range(0, len(d), 2)]
                    new_a.append(a[k] + d[0])
                return (carry[0],) + tuple(new_a)

            def slow(carry):
                return lax.fori_loop(
                    0, LANES, lambda i, cc: row_body(g * LANES + i, cc), carry
                )

            return lax.cond(n_same == LANES, fast, slow, carry)

        carry = lax.fori_loop(0, nb * 8 // LANES, group_body, carry)

    prev = carry[0]
    for k in range(NVEC):
        acc[pl.ds(prev * HALF + k * LANES, LANES)] = carry[1 + k]

    @pl.when(q == 0)
    def _():
        pltpu.sync_copy(
            x4_hbm.at[pl.ds(WPH * RPB, XTRA_B), h],
            rows_a.at[pl.ds(0, XTRA_B)],
        )

        def tail_row(r, carry):
            prev = carry[0]
            a = carry[1:]
            seg = idx_v[pl.ds(IDXT + r, LANES)][0]
            flush = seg != prev

            @pl.when(flush)
            def _():
                for k in range(NVEC):
                    sl = pl.ds(prev * HALF + k * LANES, LANES)
                    acc[sl] = acc[sl] + a[k]

            fv = jnp.full((LANES,), flush)
            new_a = tuple(
                jnp.where(
                    fv,
                    rows_a[r // 8, lax.rem(r, 8), pl.ds(k * LANES, LANES)],
                    a[k] + rows_a[r // 8, lax.rem(r, 8), pl.ds(k * LANES, LANES)],
                )
                for k in range(NVEC)
            )
            return (seg,) + new_a

        tcarry = (idx_v[pl.ds(IDXT, LANES)][0],) + zero8
        tcarry = lax.fori_loop(0, XTRA, tail_row, tcarry)
        tprev = tcarry[0]
        for k in range(NVEC):
            sl = pl.ds(tprev * HALF + k * LANES, LANES)
            acc[sl] = acc[sl] + tcarry[1 + k]

    pltpu.sync_copy(acc, out_hbm.at[c].at[s])


def _segment_sum_sc(x4, idx):
    mesh = plsc.VectorSubcoreMesh(
        core_axis_name="c", subcore_axis_name="s", num_cores=NC, num_subcores=NS
    )
    return pl.kernel(
        _sc_body,
        out_type=jax.ShapeDtypeStruct((NC, NS, GRAPHS * HALF), jnp.float32),
        mesh=mesh,
        compiler_params=pltpu.CompilerParams(
            use_tc_tiling_on_sc=False, needs_layout_passes=False
        ),
        scratch_types=[
            pltpu.VMEM((RPW + XTRA + LANES,), jnp.int32),
            pltpu.VMEM((BCH, 8, HALF), jnp.float32),
            pltpu.VMEM((BCH, 8, HALF), jnp.float32),
            pltpu.VMEM((GRAPHS * HALF,), jnp.float32),
            pltpu.SemaphoreType.DMA,
            pltpu.SemaphoreType.DMA,
        ],
    )(x4, idx)


def _tc_body(p_ref, w_ref, b_ref, o_ref):
    p = p_ref[...].reshape(NC, NS, GRAPHS, HALF)
    lo = jnp.sum(p[:, 0:8], axis=(0, 1))
    hi = jnp.sum(p[:, 8:16], axis=(0, 1))
    pooled = jnp.concatenate([lo, hi], axis=1)
    o_ref[...] = (
        lax.dot_general(
            pooled, w_ref[...], (((1,), (1,)), ((), ())),
            preferred_element_type=jnp.float32,
        )
        + b_ref[...]
    )


def _linear_tc(parts, W, b):
    return pl.pallas_call(
        _tc_body,
        out_shape=jax.ShapeDtypeStruct((GRAPHS, OUT), jnp.float32),
    )(parts, W, b.reshape(1, OUT))


def kernel(x, batch, W, b):
    idx = batch.astype(jnp.int32)
    x4 = jnp.transpose(x.reshape(NBLK, 8, 2, HALF), (0, 2, 1, 3))
    parts = _segment_sum_sc(x4, idx)
    return _linear_tc(parts, W, b)


# speedup vs baseline: 6.5412x; 1.5807x over previous
import jax
import jax.numpy as jnp
from jax import lax
from jax.experimental import pallas as pl
from jax.experimental.pallas import tpu as pltpu
from jax.experimental.pallas import tpu_sc as plsc

N_NODES = 50000
HIDDEN = 256
GRAPHS = 512
OUT = 128

NC = 2
NS = 16
NW = NC * NS
NBLK = N_NODES // 8
RPB = 195
RPW = RPB * 8
BCH = 13
NCHUNK = RPB // BCH
CROWS = BCH * 8
XTRA_B = NBLK - NW * RPB
XTRA = XTRA_B * 8
LANES = 16
NVEC = HIDDEN // LANES
SLOTS = GRAPHS + NW
SBATCH = 64
TIDX = 1568
PIDX = 1656


def _sc_body(x4_hbm, idx_hbm, out_hbm, idx_v, rows_a, rows_b, stg, tmp,
             sem_a, sem_b):
    c = lax.axis_index("c")
    s = lax.axis_index("s")
    w = s * NC + c

    pltpu.sync_copy(idx_hbm.at[pl.ds(w * RPW, RPW)], idx_v.at[pl.ds(0, RPW)])
    pltpu.sync_copy(
        idx_hbm.at[pl.ds(NW * RPW, XTRA)], idx_v.at[pl.ds(TIDX, XTRA)]
    )

    @pl.when(w != 0)
    def _():
        pltpu.sync_copy(
            idx_hbm.at[pl.ds(w * RPW - 8, 8)], idx_v.at[pl.ds(PIDX, 8)]
        )

    prev_init = jnp.where(w == 0, 0, idx_v[pl.ds(PIDX, LANES)][7])
    span = prev_init + w

    bufs = (rows_a, rows_b)
    sems = (sem_a, sem_b)

    def fetch(j):
        src = x4_hbm.at[pl.ds(w * RPB + j * BCH, BCH)]
        return pltpu.async_copy(src, bufs[j % 2], sems[j % 2])

    zvec = jnp.zeros((LANES,), jnp.float32)
    zero16 = (zvec,) * NVEC

    def emit_row(vecs, fc):
        si = lax.rem(fc, SBATCH)
        for k in range(NVEC):
            stg[pl.ds(si * HIDDEN + k * LANES, LANES)] = vecs[k]
        fc2 = fc + 1

        @pl.when(lax.rem(fc2, SBATCH) == 0)
        def _():
            pltpu.sync_copy(
                stg,
                out_hbm.at[pl.ds((span + fc2 - SBATCH) * HIDDEN,
                                 SBATCH * HIDDEN)],
            )

        return fc2

    def gap_body(i, fc):
        return emit_row(zero16, fc)

    carry = (prev_init, jnp.int32(0)) + zero16

    pending = fetch(0)
    for j in range(NCHUNK):
        start = j * CROWS
        rows_v = bufs[j % 2]
        nxt = fetch(j + 1) if j < NCHUNK - 1 else None
        pending.wait()
        pending = nxt

        def row_body(r, carry):
            prev = carry[0]
            seg = idx_v[pl.ds(start + r, LANES)][0]
            b = r // 8
            ri = lax.rem(r, 8)
            d = tuple(
                rows_v[b, k // 8, ri, pl.ds((k % 8) * LANES, LANES)]
                for k in range(NVEC)
            )

            def flushed(ops):
                fc = emit_row(ops[2:], ops[1])
                fc = lax.fori_loop(0, seg - ops[0] - 1, gap_body, fc)
                return (seg, fc) + d

            def kept(ops):
                return (seg, ops[1]) + tuple(
                    aa + dd for aa, dd in zip(ops[2:], d)
                )

            return lax.cond(seg != prev, flushed, kept, carry)

        def group_body(g, carry):
            prev = carry[0]
            seg_vec = idx_v[pl.ds(start + g * LANES, LANES)]
            n_same = plsc.all_reduce_population_count(
                seg_vec == jnp.full((LANES,), prev, jnp.int32)
            )[0]

            def fast(ops):
                def frow(i, a):
                    r = g * LANES + i
                    b = r // 8
                    ri = lax.rem(r, 8)
                    return tuple(
                        a[k]
                        + rows_v[b, k // 8, ri, pl.ds((k % 8) * LANES, LANES)]
                        for k in range(NVEC)
                    )

                return ops[:2] + lax.fori_loop(0, LANES, frow, ops[2:])

            def slow(ops):
                return lax.fori_loop(
                    0, LANES, lambda i, cc: row_body(g * LANES + i, cc), ops
                )

            return lax.cond(n_same == LANES, fast, slow, carry)

        carry = lax.fori_loop(0, CROWS // LANES, group_body, carry)

    fc = emit_row(carry[2:], carry[1])
    fc = lax.cond(
        w == NW - 1,
        lambda f: lax.fori_loop(0, SLOTS - span - f, gap_body, f),
        lambda f: f,
        fc,
    )

    res = lax.rem(fc, SBATCH)
    base = span + fc - res
    off = jnp.int32(0)
    for sz in (32, 16, 8, 4, 2, 1):
        take = (res & sz) != 0
        cur = off

        @pl.when(take)
        def _(sz=sz, cur=cur):
            pltpu.sync_copy(
                stg.at[pl.ds(cur * HIDDEN, sz * HIDDEN)],
                out_hbm.at[pl.ds((base + cur) * HIDDEN, sz * HIDDEN)],
            )

        off = off + jnp.where(take, sz, 0)

    @pl.when(w == NW - 1)
    def _():
        pltpu.sync_copy(
            x4_hbm.at[pl.ds(NW * RPB, XTRA_B)], rows_a.at[pl.ds(0, XTRA_B)]
        )

        def rmw(slot, vecs):
            src = out_hbm.at[pl.ds(slot * HIDDEN, HIDDEN)]
            pltpu.sync_copy(src, tmp)
            for k in range(NVEC):
                sl = pl.ds(k * LANES, LANES)
                tmp[sl] = tmp[sl] + vecs[k]
            pltpu.sync_copy(tmp, out_hbm.at[pl.ds(slot * HIDDEN, HIDDEN)])

        def tail_row(r, carry):
            prev = carry[0]
            seg = idx_v[pl.ds(TIDX + r, LANES)][0]
            b = r // 8
            ri = lax.rem(r, 8)
            d = tuple(
                rows_a[b, k // 8, ri, pl.ds((k % 8) * LANES, LANES)]
                for k in range(NVEC)
            )

            def flushed(ops):
                rmw(ops[0] + NW - 1, ops[1:])
                return (seg,) + d

            def kept(ops):
                return (seg,) + tuple(aa + dd for aa, dd in zip(ops[1:], d))

            return lax.cond(seg != prev, flushed, kept, carry)

        tcarry = (idx_v[pl.ds(TIDX, LANES)][0],) + zero16
        tcarry = lax.fori_loop(0, XTRA, tail_row, tcarry)
        rmw(tcarry[0] + NW - 1, tcarry[1:])


def _segment_sum_sc(x4, idx):
    mesh = plsc.VectorSubcoreMesh(
        core_axis_name="c", subcore_axis_name="s", num_cores=NC, num_subcores=NS
    )
    return pl.kernel(
        _sc_body,
        out_type=jax.ShapeDtypeStruct((SLOTS * HIDDEN,), jnp.float32),
        mesh=mesh,
        compiler_params=pltpu.CompilerParams(
            use_tc_tiling_on_sc=False, needs_layout_passes=False
        ),
        scratch_types=[
            pltpu.VMEM((1680,), jnp.int32),
            pltpu.VMEM((BCH, 2, 8, 128), jnp.float32),
            pltpu.VMEM((BCH, 2, 8, 128), jnp.float32),
            pltpu.VMEM((SBATCH * HIDDEN,), jnp.float32),
            pltpu.VMEM((HIDDEN,), jnp.float32),
            pltpu.SemaphoreType.DMA,
            pltpu.SemaphoreType.DMA,
        ],
    )(x4, idx)


def _tc_body(p_ref, w_ref, b_ref, o_ref):
    t = p_ref[...]
    for sh in (1, 2, 4, 8, 16):
        t = t[: t.shape[0] - sh] + t[sh:]
    pooled = t[:GRAPHS]
    o_ref[...] = (
        lax.dot_general(
            pooled, w_ref[...], (((1,), (1,)), ((), ())),
            preferred_element_type=jnp.float32,
        )
        + b_ref[...]
    )


def _linear_tc(compact, W, b):
    return pl.pallas_call(
        _tc_body,
        out_shape=jax.ShapeDtypeStruct((GRAPHS, OUT), jnp.float32),
    )(compact, W, b.reshape(1, OUT))


def kernel(x, batch, W, b):
    idx = batch.astype(jnp.int32)
    x4 = jnp.transpose(x.reshape(NBLK, 8, 2, 128), (0, 2, 1, 3))
    compact = _segment_sum_sc(x4, idx).reshape(SLOTS, HIDDEN)
    return _linear_tc(compact, W, b)
